# baseline (speedup 1.0000x reference)
import jax
import jax.numpy as jnp
from jax import lax
from jax.experimental import pallas as pl
from jax.experimental.pallas import tpu as pltpu
from jax.experimental.pallas import tpu_sc as plsc

N_NODES = 10000
D = 128
E_TOTAL = 320000
NC, NS = 2, 16
NW = NC * NS
E_PER_W = 10240
E_PAD = E_PER_W * NW
CHUNK = 128
NCHUNK = E_PER_W // CHUNK
N_PAD = 10240
ROWS_PER_TILE = N_PAD // NS
TC_BLK = 1000


def _sc_body(rows_hbm, cols_hbm, vals_hbm, x_hbm, zero_hbm, part_hbm,
             cols2d, rowv0, rowv1, valv0, valv1, gbuf0, gbuf1, acc_sh,
             gsem0, gsem1, ssem0, ssem1):
    c = lax.axis_index("c")
    s = lax.axis_index("s")
    wid = s * NC + c

    r0 = s * ROWS_PER_TILE
    pltpu.sync_copy(zero_hbm.at[pl.ds(r0, ROWS_PER_TILE)],
                    acc_sh.at[pl.ds(r0, ROWS_PER_TILE)])
    pltpu.sync_copy(cols_hbm.at[wid], cols2d)
    plsc.subcore_barrier()

    gbufs = (gbuf0, gbuf1)
    rowvs = (rowv0, rowv1)
    valvs = (valv0, valv1)
    sems = (gsem0, gsem1)
    ssems = (ssem0, ssem1)

    def issue(ci, b):
        pltpu.async_copy(x_hbm.at[cols2d.at[ci]], gbufs[b], sems[b])
        pltpu.async_copy(rows_hbm.at[wid, ci], rowvs[b], sems[b])
        pltpu.async_copy(vals_hbm.at[wid, ci], valvs[b], sems[b])

    def drain(ci, b):
        pltpu.make_async_copy(x_hbm.at[cols2d.at[ci]], gbufs[b], sems[b]).wait()
        pltpu.make_async_copy(rows_hbm.at[wid, ci], rowvs[b], sems[b]).wait()
        pltpu.make_async_copy(vals_hbm.at[wid, ci], valvs[b], sems[b]).wait()

    def scale_scatter(b):
        gb = gbufs[b]

        def group_body(gi, carry2):
            vv = valvs[b][pl.ds(gi * 16, 16)]
            for l in range(16):
                v = jnp.full((16,), vv[l], jnp.float32)
                e = gi * 16 + l
                for j in range(D // 16):
                    g = gb[e, pl.ds(j * 16, 16)]
                    gb[e, pl.ds(j * 16, 16)] = g * v
            return carry2

        lax.fori_loop(0, CHUNK // 16, group_body, 0)

    issue(0, 0)

    def pipe_body(pi, carry):
        for b in range(2):
            ci = pi * 2 + b
            drain(ci, b)

            @pl.when(ci >= 1)
            def _drain_scatter():
                pltpu.make_async_copy(
                    gbufs[1 - b], acc_sh.at[rowvs[1 - b]], ssems[1 - b]).wait()

            @pl.when(ci + 1 < NCHUNK)
            def _prefetch():
                issue(ci + 1, 1 - b)

            scale_scatter(b)
            pltpu.async_copy(gbufs[b], acc_sh.at[rowvs[b]], ssems[b],
                             add=True)
        return carry

    lax.fori_loop(0, NCHUNK // 2, pipe_body, 0)
    pltpu.make_async_copy(gbufs[1], acc_sh.at[rowvs[1]], ssems[1]).wait()
    plsc.subcore_barrier()
    pltpu.sync_copy(acc_sh.at[pl.ds(r0, ROWS_PER_TILE)],
                    part_hbm.at[c, pl.ds(r0, ROWS_PER_TILE)])


def _make_sc_scatter():
    return pl.kernel(
        _sc_body,
        out_type=jax.ShapeDtypeStruct((NC, N_PAD, D), jnp.float32),
        mesh=plsc.VectorSubcoreMesh(core_axis_name="c", subcore_axis_name="s",
                                    num_cores=NC, num_subcores=NS),
        scratch_types=[
            pltpu.VMEM((NCHUNK, CHUNK), jnp.int32),
            pltpu.VMEM((CHUNK,), jnp.int32),
            pltpu.VMEM((CHUNK,), jnp.int32),
            pltpu.VMEM((CHUNK,), jnp.float32),
            pltpu.VMEM((CHUNK,), jnp.float32),
            pltpu.VMEM((CHUNK, D), jnp.float32),
            pltpu.VMEM((CHUNK, D), jnp.float32),
            pltpu.VMEM_SHARED((N_PAD, D), jnp.float32),
            pltpu.SemaphoreType.DMA,
            pltpu.SemaphoreType.DMA,
            pltpu.SemaphoreType.DMA,
            pltpu.SemaphoreType.DMA,
        ],
    )


def _tc_body(p_ref, w_ref, b_ref, o_ref):
    acc = p_ref[0] + p_ref[1]
    y = jnp.dot(acc, w_ref[...], preferred_element_type=jnp.float32,
                precision=lax.Precision.HIGHEST)
    o_ref[...] = jnp.maximum(y + b_ref[...], 0.0)


_tc_finish = pl.pallas_call(
    _tc_body,
    grid=(N_NODES // TC_BLK,),
    in_specs=[
        pl.BlockSpec((NC, TC_BLK, D), lambda i: (0, i, 0)),
        pl.BlockSpec((D, D), lambda i: (0, 0)),
        pl.BlockSpec((1, D), lambda i: (0, 0)),
    ],
    out_specs=pl.BlockSpec((TC_BLK, D), lambda i: (i, 0)),
    out_shape=jax.ShapeDtypeStruct((N_NODES, D), jnp.float32),
)


def kernel(x, edge_index, edge_vals, W, b, num_features_nonzero):
    pad = E_PAD - E_TOTAL
    rows = jnp.concatenate(
        [edge_index[0].astype(jnp.int32),
         jnp.full((pad,), N_NODES, jnp.int32)]).reshape(NW, NCHUNK, CHUNK)
    cols = jnp.concatenate(
        [edge_index[1].astype(jnp.int32),
         jnp.zeros((pad,), jnp.int32)]).reshape(NW, NCHUNK, CHUNK)
    vals = jnp.concatenate(
        [edge_vals.astype(jnp.float32),
         jnp.zeros((pad,), jnp.float32)]).reshape(NW, NCHUNK, CHUNK)
    x = x.astype(jnp.float32)
    zeros = jnp.zeros((N_PAD, D), jnp.float32)
    part = _make_sc_scatter()(rows, cols, vals, x, zeros)
    return _tc_finish(part, W.astype(jnp.float32), b.reshape(1, D))


# speedup vs baseline: 3.2425x; 3.2425x over previous
import jax
import jax.numpy as jnp
from jax import lax
from jax.experimental import pallas as pl
from jax.experimental.pallas import tpu as pltpu
from jax.experimental.pallas import tpu_sc as plsc

N_NODES = 10000
D = 128
E_TOTAL = 320000
NC, NS = 2, 16
NW = NC * NS
E_PER_W = 10240
E_PAD = E_PER_W * NW
CHUNK = 128
NCHUNK = E_PER_W // CHUNK
N_PAD = 10240
ROWS_PER_TILE = N_PAD // NS
TC_BLK = 1000


def _sc_body(rows_hbm, cols_hbm, vals_hbm, x_hbm, zero_hbm, part_hbm,
             cols2d, rowv0, rowv1, valv0, valv1, gbuf0, gbuf1, acc_sh,
             gsem0, gsem1, ssem0, ssem1):
    c = lax.axis_index("c")
    s = lax.axis_index("s")
    wid = s * NC + c

    r0 = s * ROWS_PER_TILE
    pltpu.sync_copy(zero_hbm.at[pl.ds(r0, ROWS_PER_TILE)],
                    acc_sh.at[pl.ds(r0, ROWS_PER_TILE)])
    pltpu.sync_copy(cols_hbm.at[wid], cols2d)
    plsc.subcore_barrier()

    gbufs = (gbuf0, gbuf1)
    rowvs = (rowv0, rowv1)
    valvs = (valv0, valv1)
    sems = (gsem0, gsem1)
    ssems = (ssem0, ssem1)

    def issue(ci, b):
        pltpu.async_copy(x_hbm.at[cols2d.at[ci]], gbufs[b], sems[b])
        pltpu.async_copy(rows_hbm.at[wid, ci], rowvs[b], sems[b])
        pltpu.async_copy(vals_hbm.at[wid, ci], valvs[b], sems[b])

    def drain(ci, b):
        pltpu.make_async_copy(x_hbm.at[cols2d.at[ci]], gbufs[b], sems[b]).wait()
        pltpu.make_async_copy(rows_hbm.at[wid, ci], rowvs[b], sems[b]).wait()
        pltpu.make_async_copy(vals_hbm.at[wid, ci], valvs[b], sems[b]).wait()

    def scale_scatter(b):
        gb = gbufs[b]

        @plsc.parallel_loop(0, CHUNK // 16, unroll=2)
        def group_body(gi):
            vv = valvs[b][pl.ds(gi * 16, 16)]
            for l in range(16):
                v = jnp.full((16,), vv[l], jnp.float32)
                e = gi * 16 + l
                for j in range(D // 16):
                    g = gb[e, pl.ds(j * 16, 16)]
                    gb[e, pl.ds(j * 16, 16)] = g * v

    issue(0, 0)

    def pipe_body(pi, carry):
        for b in range(2):
            ci = pi * 2 + b
            drain(ci, b)

            @pl.when(ci >= 1)
            def _drain_scatter():
                pltpu.make_async_copy(
                    gbufs[1 - b], acc_sh.at[rowvs[1 - b]], ssems[1 - b]).wait()

            @pl.when(ci + 1 < NCHUNK)
            def _prefetch():
                issue(ci + 1, 1 - b)

            scale_scatter(b)
            pltpu.async_copy(gbufs[b], acc_sh.at[rowvs[b]], ssems[b],
                             add=True)
        return carry

    lax.fori_loop(0, NCHUNK // 2, pipe_body, 0)
    pltpu.make_async_copy(gbufs[1], acc_sh.at[rowvs[1]], ssems[1]).wait()
    plsc.subcore_barrier()
    pltpu.sync_copy(acc_sh.at[pl.ds(r0, ROWS_PER_TILE)],
                    part_hbm.at[c, pl.ds(r0, ROWS_PER_TILE)])


def _make_sc_scatter():
    return pl.kernel(
        _sc_body,
        out_type=jax.ShapeDtypeStruct((NC, N_PAD, D), jnp.float32),
        mesh=plsc.VectorSubcoreMesh(core_axis_name="c", subcore_axis_name="s",
                                    num_cores=NC, num_subcores=NS),
        scratch_types=[
            pltpu.VMEM((NCHUNK, CHUNK), jnp.int32),
            pltpu.VMEM((CHUNK,), jnp.int32),
            pltpu.VMEM((CHUNK,), jnp.int32),
            pltpu.VMEM((CHUNK,), jnp.float32),
            pltpu.VMEM((CHUNK,), jnp.float32),
            pltpu.VMEM((CHUNK, D), jnp.float32),
            pltpu.VMEM((CHUNK, D), jnp.float32),
            pltpu.VMEM_SHARED((N_PAD, D), jnp.float32),
            pltpu.SemaphoreType.DMA,
            pltpu.SemaphoreType.DMA,
            pltpu.SemaphoreType.DMA,
            pltpu.SemaphoreType.DMA,
        ],
    )


def _tc_body(p_ref, w_ref, b_ref, o_ref):
    acc = p_ref[0] + p_ref[1]
    y = jnp.dot(acc, w_ref[...], preferred_element_type=jnp.float32,
                precision=lax.Precision.HIGHEST)
    o_ref[...] = jnp.maximum(y + b_ref[...], 0.0)


_tc_finish = pl.pallas_call(
    _tc_body,
    grid=(N_NODES // TC_BLK,),
    in_specs=[
        pl.BlockSpec((NC, TC_BLK, D), lambda i: (0, i, 0)),
        pl.BlockSpec((D, D), lambda i: (0, 0)),
        pl.BlockSpec((1, D), lambda i: (0, 0)),
    ],
    out_specs=pl.BlockSpec((TC_BLK, D), lambda i: (i, 0)),
    out_shape=jax.ShapeDtypeStruct((N_NODES, D), jnp.float32),
)


def kernel(x, edge_index, edge_vals, W, b, num_features_nonzero):
    pad = E_PAD - E_TOTAL
    pad_rows = N_NODES + (jnp.arange(pad, dtype=jnp.int32) % (N_PAD - N_NODES))
    pad_cols = jnp.arange(pad, dtype=jnp.int32) % N_NODES
    rows = jnp.concatenate(
        [edge_index[0].astype(jnp.int32), pad_rows]).reshape(NW, NCHUNK, CHUNK)
    cols = jnp.concatenate(
        [edge_index[1].astype(jnp.int32), pad_cols]).reshape(NW, NCHUNK, CHUNK)
    vals = jnp.concatenate(
        [edge_vals.astype(jnp.float32),
         jnp.zeros((pad,), jnp.float32)]).reshape(NW, NCHUNK, CHUNK)
    x = x.astype(jnp.float32)
    zeros = jnp.zeros((N_PAD, D), jnp.float32)
    part = _make_sc_scatter()(rows, cols, vals, x, zeros)
    return _tc_finish(part, W.astype(jnp.float32), b.reshape(1, D))


# speedup vs baseline: 3.3910x; 1.0458x over previous
import jax
import jax.numpy as jnp
from jax import lax
from jax.experimental import pallas as pl
from jax.experimental.pallas import tpu as pltpu
from jax.experimental.pallas import tpu_sc as plsc

N_NODES = 10000
D = 128
E_TOTAL = 320000
NC, NS = 2, 16
NW = NC * NS
E_PER_W = E_TOTAL // NW
CHUNK = 128
NFULL = E_PER_W // CHUNK
TAIL = E_PER_W - NFULL * CHUNK
N_PAD = 10240
ROWS_PER_TILE = N_PAD // NS
TC_BLK = 1000


def _sc_body(rows_hbm, cols_hbm, vals_hbm, x_hbm, part_hbm,
             colsv, rowv0, rowv1, valv0, valv1, rowv_t, valv_t,
             gbuf0, gbuf1, acc_sh, gsem0, gsem1, ssem0, ssem1):
    c = lax.axis_index("c")
    s = lax.axis_index("s")
    wid = s * NC + c
    ebase = wid * E_PER_W
    r0 = s * ROWS_PER_TILE

    @plsc.parallel_loop(0, CHUNK)
    def _zrow(i):
        for j in range(D // 16):
            gbuf0[i, pl.ds(j * 16, 16)] = jnp.zeros((16,), jnp.float32)

    for i in range(ROWS_PER_TILE // CHUNK):
        pltpu.sync_copy(gbuf0, acc_sh.at[pl.ds(r0 + i * CHUNK, CHUNK)])
    pltpu.sync_copy(cols_hbm.at[pl.ds(ebase, E_PER_W)], colsv)
    plsc.subcore_barrier()

    gbufs = (gbuf0, gbuf1)
    rowvs = (rowv0, rowv1)
    valvs = (valv0, valv1)
    sems = (gsem0, gsem1)
    ssems = (ssem0, ssem1)

    def issue(ci, b):
        base = ebase + ci * CHUNK
        pltpu.async_copy(x_hbm.at[colsv.at[pl.ds(ci * CHUNK, CHUNK)]],
                         gbufs[b], sems[b])
        pltpu.async_copy(rows_hbm.at[pl.ds(base, CHUNK)], rowvs[b], sems[b])
        pltpu.async_copy(vals_hbm.at[pl.ds(base, CHUNK)], valvs[b], sems[b])

    def drain(ci, b):
        base = ebase + ci * CHUNK
        pltpu.make_async_copy(x_hbm.at[colsv.at[pl.ds(ci * CHUNK, CHUNK)]],
                              gbufs[b], sems[b]).wait()
        pltpu.make_async_copy(rows_hbm.at[pl.ds(base, CHUNK)], rowvs[b],
                              sems[b]).wait()
        pltpu.make_async_copy(vals_hbm.at[pl.ds(base, CHUNK)], valvs[b],
                              sems[b]).wait()

    def scale(b):
        gb = gbufs[b]

        @plsc.parallel_loop(0, CHUNK // 16, unroll=2)
        def group_body(gi):
            vv = valvs[b][pl.ds(gi * 16, 16)]
            for l in range(16):
                v = jnp.full((16,), vv[l], jnp.float32)
                e = gi * 16 + l
                for j in range(D // 16):
                    g = gb[e, pl.ds(j * 16, 16)]
                    gb[e, pl.ds(j * 16, 16)] = g * v

    issue(0, 0)

    def pipe_body(pi, carry):
        for b in range(2):
            ci = pi * 2 + b
            drain(ci, b)

            @pl.when(ci >= 1)
            def _drain_scatter():
                pltpu.make_async_copy(
                    gbufs[1 - b], acc_sh.at[rowvs[1 - b]], ssems[1 - b]).wait()

            @pl.when(ci + 1 < NFULL)
            def _prefetch():
                issue(ci + 1, 1 - b)

            scale(b)
            pltpu.async_copy(gbufs[b], acc_sh.at[rowvs[b]], ssems[b],
                             add=True)
        return carry

    lax.fori_loop(0, NFULL // 2, pipe_body, 0)
    pltpu.make_async_copy(gbufs[1], acc_sh.at[rowvs[1]], ssems[1]).wait()

    tbase = ebase + NFULL * CHUNK
    pltpu.sync_copy(rows_hbm.at[pl.ds(tbase, TAIL)], rowv_t)
    pltpu.sync_copy(vals_hbm.at[pl.ds(tbase, TAIL)], valv_t)
    pltpu.async_copy(x_hbm.at[colsv.at[pl.ds(NFULL * CHUNK, TAIL)]],
                     gbuf0.at[pl.ds(0, TAIL)], gsem0).wait()
    vv = valv_t[...]
    for l in range(TAIL):
        v = jnp.full((16,), vv[l], jnp.float32)
        for j in range(D // 16):
            g = gbuf0[l, pl.ds(j * 16, 16)]
            gbuf0[l, pl.ds(j * 16, 16)] = g * v
    pltpu.sync_copy(gbuf0.at[pl.ds(0, TAIL)], acc_sh.at[rowv_t], add=True)

    plsc.subcore_barrier()
    pltpu.sync_copy(acc_sh.at[pl.ds(r0, ROWS_PER_TILE)],
                    part_hbm.at[c, pl.ds(r0, ROWS_PER_TILE)])


def _make_sc_scatter():
    return pl.kernel(
        _sc_body,
        out_type=jax.ShapeDtypeStruct((NC, N_PAD, D), jnp.float32),
        mesh=plsc.VectorSubcoreMesh(core_axis_name="c", subcore_axis_name="s",
                                    num_cores=NC, num_subcores=NS),
        scratch_types=[
            pltpu.VMEM((E_PER_W,), jnp.int32),
            pltpu.VMEM((CHUNK,), jnp.int32),
            pltpu.VMEM((CHUNK,), jnp.int32),
            pltpu.VMEM((CHUNK,), jnp.float32),
            pltpu.VMEM((CHUNK,), jnp.float32),
            pltpu.VMEM((TAIL,), jnp.int32),
            pltpu.VMEM((TAIL,), jnp.float32),
            pltpu.VMEM((CHUNK, D), jnp.float32),
            pltpu.VMEM((CHUNK, D), jnp.float32),
            pltpu.VMEM_SHARED((N_PAD, D), jnp.float32),
            pltpu.SemaphoreType.DMA,
            pltpu.SemaphoreType.DMA,
            pltpu.SemaphoreType.DMA,
            pltpu.SemaphoreType.DMA,
        ],
    )


def _tc_body(p_ref, w_ref, b_ref, o_ref):
    acc = p_ref[0] + p_ref[1]
    y = jnp.dot(acc, w_ref[...], preferred_element_type=jnp.float32,
                precision=lax.Precision.HIGHEST)
    o_ref[...] = jnp.maximum(y + b_ref[...], 0.0)


_tc_finish = pl.pallas_call(
    _tc_body,
    grid=(N_NODES // TC_BLK,),
    in_specs=[
        pl.BlockSpec((NC, TC_BLK, D), lambda i: (0, i, 0)),
        pl.BlockSpec((D, D), lambda i: (0, 0)),
        pl.BlockSpec((1, D), lambda i: (0, 0)),
    ],
    out_specs=pl.BlockSpec((TC_BLK, D), lambda i: (i, 0)),
    out_shape=jax.ShapeDtypeStruct((N_NODES, D), jnp.float32),
)


def kernel(x, edge_index, edge_vals, W, b, num_features_nonzero):
    ei = edge_index.astype(jnp.int32)
    part = _make_sc_scatter()(ei[0], ei[1], edge_vals.astype(jnp.float32),
                              x.astype(jnp.float32))
    return _tc_finish(part, W.astype(jnp.float32), b.reshape(1, D))
